# Initial kernel scaffold; baseline (speedup 1.0000x reference)
import jax
import jax.numpy as jnp
from jax.experimental import pallas as pl


def kernel(q_seed_features, q_seed_labels, t_seed_features, t_seed_labels, enc_inds_q, enc_inds_t, cl_loss_label, pc_q, pc_t):
    raise NotImplementedError("write your pallas kernel here")


# speedup vs baseline: 29.4498x; 29.4498x over previous
import functools

import numpy as np
import jax
import jax.numpy as jnp
from jax import lax
from jax.experimental import pallas as pl
from jax.experimental.pallas import tpu as pltpu
from jax.experimental.pallas import tpu_sc as plsc

B, N, D = 4, 1024, 128
NPOS = 64
TEMP = 0.07

_rp = np.zeros(N, dtype=np.int64)
_rn = np.zeros((N, NPOS - 1), dtype=np.int64)
for _m in range(1, N):
    np.random.seed(0)
    _rp[_m] = np.random.choice(_m, 1)[0]
    _k = N - _m
    _rn[_m] = np.random.choice(_k, NPOS - 1, replace=_k < NPOS - 1)
_COMB = np.zeros((N, 128), dtype=np.float32)
_COMB[:, : NPOS - 1] = _rn
_COMB[:, NPOS - 1] = _rp


def _stage1_body(qf_ref, tf_ref, qlc_ref, tl3_ref, tlc_ref, qlr_ref, tab_ref,
                 s_ref, fidx_ref, ok_ref):
    b = pl.program_id(0)
    qf = qf_ref[0]
    tf = tf_ref[0]
    qlc = qlc_ref[0][:, 0:1]
    tl = tl3_ref[0]
    tlc = tlc_ref[0][:, 0:1]
    qlr = qlr_ref[0]

    qn = qf / jnp.maximum(jnp.sqrt(jnp.sum(qf * qf, axis=1, keepdims=True)), 1e-12)
    tn = tf / jnp.maximum(jnp.sqrt(jnp.sum(tf * tf, axis=1, keepdims=True)), 1e-12)
    s_ref[0] = lax.dot_general(qn, tn, (((1,), (1,)), ((), ())),
                               preferred_element_type=jnp.float32,
                               precision=lax.Precision.HIGHEST)

    mask = qlc == tl
    mf = jnp.sum(mask.astype(jnp.float32), axis=1, keepdims=True)
    okv = jnp.where((mf >= 1.0) & (mf <= float(N - 1)), 1.0, 0.0)
    ok_ref[0] = jnp.full((1, 128), jnp.min(okv), jnp.float32)

    mc = jnp.clip(mf, 1.0, float(N - 1)).astype(jnp.int32)
    vio = lax.broadcasted_iota(jnp.int32, (NPOS, N), 1)
    oh = (mc == vio).astype(jnp.float32)
    comb = lax.dot_general(oh, tab_ref[...], (((1,), (0,)), ((), ())),
                           preferred_element_type=jnp.float32,
                           precision=lax.Precision.HIGHEST)
    combi = jnp.round(comb).astype(jnp.int32)

    rn64 = combi[:, 0:NPOS]
    rp_col = combi[:, NPOS - 1:NPOS]
    kk = lax.broadcasted_iota(jnp.int32, (NPOS, NPOS), 1)
    pp = lax.broadcasted_iota(jnp.int32, (NPOS, NPOS), 0)
    shift = jnp.concatenate(
        [jnp.zeros((NPOS, 1), jnp.int32), rn64[:, 0:NPOS - 1]], axis=1)
    tslot = jnp.where(kk == pp, rp_col + 1,
                      N + 1 + jnp.where(kk < pp, rn64, shift))

    mt = (tlc == qlr).astype(jnp.float32)
    ioq = lax.broadcasted_iota(jnp.int32, (N, N), 0)
    ioq2 = lax.broadcasted_iota(jnp.int32, (N, N), 1)
    ltri = (ioq2 <= ioq).astype(jnp.float32)
    ct = lax.dot_general(ltri, mt, (((1,), (0,)), ((), ())),
                         preferred_element_type=jnp.float32,
                         precision=lax.Precision.HIGHEST)
    cti = jnp.round(ct).astype(jnp.int32)

    qcol = lax.broadcasted_iota(jnp.int32, (N, NPOS), 0)
    slott = jnp.where(mt > 0.5, cti, N + qcol + 1 - cti)

    qrow = lax.broadcasted_iota(jnp.int32, (1, N), 1).astype(jnp.float32)
    for p in range(NPOS):
        matcht = (slott[:, p:p + 1] == tslot[p:p + 1, :]).astype(jnp.float32)
        qsel = lax.dot_general(qrow, matcht, (((1,), (0,)), ((), ())),
                               preferred_element_type=jnp.float32,
                               precision=lax.Precision.HIGHEST)
        fidx_ref[0, p:p + 1, :] = (
            jnp.round(qsel).astype(jnp.int32) + (b * NPOS + p) * N)


def _stage1(qf, tf, qlc, tl3, tlc, qlr, tab):
    return pl.pallas_call(
        _stage1_body,
        grid=(B,),
        in_specs=[
            pl.BlockSpec((1, NPOS, D), lambda b: (b, 0, 0)),
            pl.BlockSpec((1, N, D), lambda b: (b, 0, 0)),
            pl.BlockSpec((1, NPOS, 128), lambda b: (b, 0, 0)),
            pl.BlockSpec((1, 1, N), lambda b: (b, 0, 0)),
            pl.BlockSpec((1, N, 128), lambda b: (b, 0, 0)),
            pl.BlockSpec((1, 1, NPOS), lambda b: (b, 0, 0)),
            pl.BlockSpec((N, 128), lambda b: (0, 0)),
        ],
        out_specs=[
            pl.BlockSpec((1, NPOS, N), lambda b: (b, 0, 0)),
            pl.BlockSpec((1, NPOS, NPOS), lambda b: (b, 0, 0)),
            pl.BlockSpec((1, 1, 128), lambda b: (b, 0, 0)),
        ],
        out_shape=[
            jax.ShapeDtypeStruct((B, NPOS, N), jnp.float32),
            jax.ShapeDtypeStruct((B, NPOS, NPOS), jnp.int32),
            jax.ShapeDtypeStruct((B, 1, 128), jnp.float32),
        ],
    )(qf, tf, qlc, tl3, tlc, qlr, tab)


def _sc_gather_body(s_hbm, fidx_hbm, out_hbm, iv, ov, sem):
    wid = lax.axis_index("s") * 2 + lax.axis_index("c")
    for t in range(8):
        task = wid * 8 + t
        pltpu.sync_copy(fidx_hbm.at[task], iv)
        pltpu.async_copy(s_hbm.at[iv], ov, sem).wait()
        pltpu.sync_copy(ov, out_hbm.at[task])


@functools.lru_cache(maxsize=1)
def _sc_gather_kernel():
    mesh = plsc.VectorSubcoreMesh(core_axis_name="c", subcore_axis_name="s")
    return pl.kernel(
        _sc_gather_body,
        mesh=mesh,
        out_type=jax.ShapeDtypeStruct((B * NPOS, NPOS), jnp.float32),
        scratch_types=[
            pltpu.VMEM((NPOS,), jnp.int32),
            pltpu.VMEM((NPOS,), jnp.float32),
            pltpu.SemaphoreType.DMA,
        ],
    )


def _stage2_body(lr_ref, ok_ref, loss_ref, out_ref):
    kk = lax.broadcasted_iota(jnp.int32, (NPOS, NPOS), 1)
    pp = lax.broadcasted_iota(jnp.int32, (NPOS, NPOS), 0)
    total = jnp.float32(0.0)
    for b in range(B):
        lr = lr_ref[b]
        okv = ok_ref[b]
        oks = jnp.max(okv)
        pred = okv[:, 0:NPOS] > 0.5
        gated = jnp.where(pred, lr * jnp.float32(1.0 / TEMP), jnp.float32(0.0))
        out_ref[b] = gated
        rmax = jnp.max(gated, axis=1, keepdims=True)
        e = jnp.exp(gated - rmax)
        lse = jnp.log(jnp.sum(e, axis=1, keepdims=True)) + rmax
        picked = jnp.sum(jnp.where(kk == pp, gated, 0.0), axis=1, keepdims=True)
        ce = jnp.sum(lse - picked) * jnp.float32(1.0 / NPOS)
        total = total + jnp.where(oks > 0.5, ce, jnp.float32(0.0))
    loss_ref[...] = jnp.full((1, 128), total, jnp.float32)


def _stage2(lraw, okf):
    return pl.pallas_call(
        _stage2_body,
        out_shape=[
            jax.ShapeDtypeStruct((1, 128), jnp.float32),
            jax.ShapeDtypeStruct((B, NPOS, NPOS), jnp.float32),
        ],
    )(lraw, okf)


def kernel(q_seed_features, q_seed_labels, t_seed_features, t_seed_labels,
           enc_inds_q, enc_inds_t, cl_loss_label, pc_q, pc_t):
    ql64 = q_seed_labels[:, :NPOS].astype(jnp.int32)
    tl = t_seed_labels.astype(jnp.int32)
    qlc = jnp.broadcast_to(ql64[:, :, None], (B, NPOS, 128))
    tlc = jnp.broadcast_to(tl[:, :, None], (B, N, 128))
    tl3 = tl.reshape(B, 1, N)
    qlr = ql64.reshape(B, 1, NPOS)
    tab = jnp.asarray(_COMB)

    s, fidx, okf = _stage1(q_seed_features, t_seed_features, qlc, tl3, tlc,
                           qlr, tab)

    lraw = _sc_gather_kernel()(s.reshape(B * NPOS * N), fidx.reshape(B * NPOS, NPOS))

    loss_v, out = _stage2(lraw.reshape(B, NPOS, NPOS), okf)
    return loss_v[0, 0], out


# speedup vs baseline: 42.2151x; 1.4335x over previous
import functools

import numpy as np
import jax
import jax.numpy as jnp
from jax import lax
from jax.experimental import pallas as pl
from jax.experimental.pallas import tpu as pltpu
from jax.experimental.pallas import tpu_sc as plsc

B, N, D = 4, 1024, 128
NPOS = 64
TEMP = 0.07

_rp = np.zeros(N, dtype=np.int64)
_rn = np.zeros((N, NPOS - 1), dtype=np.int64)
for _m in range(1, N):
    np.random.seed(0)
    _rp[_m] = np.random.choice(_m, 1)[0]
    _k = N - _m
    _rn[_m] = np.random.choice(_k, NPOS - 1, replace=_k < NPOS - 1)
_COMB = np.zeros((N, 128), dtype=np.float32)
_COMB[:, : NPOS - 1] = _rn
_COMB[:, NPOS - 1] = _rp


def _stage1_body(qf_ref, tf_ref, qlc_ref, tl3_ref, tlc_ref, qlr_ref, tab_ref,
                 s_ref, fidx_ref, ok_ref):
    b = pl.program_id(0)
    qf = qf_ref[0]
    tf = tf_ref[0]
    qlc = qlc_ref[0][:, 0:1]
    tl = tl3_ref[0]
    tlc = tlc_ref[0][:, 0:1]
    qlr = qlr_ref[0]

    qn = qf / jnp.maximum(jnp.sqrt(jnp.sum(qf * qf, axis=1, keepdims=True)), 1e-12)
    tn = tf / jnp.maximum(jnp.sqrt(jnp.sum(tf * tf, axis=1, keepdims=True)), 1e-12)
    s_ref[0] = lax.dot_general(qn, tn, (((1,), (1,)), ((), ())),
                               preferred_element_type=jnp.float32,
                               precision=lax.Precision.HIGHEST)

    mask = qlc == tl
    mf = jnp.sum(mask.astype(jnp.float32), axis=1, keepdims=True)
    okv = jnp.where((mf >= 1.0) & (mf <= float(N - 1)), 1.0, 0.0)
    ok_ref[0] = jnp.full((1, 128), jnp.min(okv), jnp.float32)

    mc = jnp.clip(mf, 1.0, float(N - 1)).astype(jnp.int32)
    vio = lax.broadcasted_iota(jnp.int32, (NPOS, N), 1)
    oh = (mc == vio).astype(jnp.float32)
    comb = lax.dot_general(oh, tab_ref[...], (((1,), (0,)), ((), ())),
                           preferred_element_type=jnp.float32,
                           precision=lax.Precision.HIGHEST)
    combi = jnp.round(comb).astype(jnp.int32)

    rn64 = combi[:, 0:NPOS]
    rp_col = combi[:, NPOS - 1:NPOS]
    kk = lax.broadcasted_iota(jnp.int32, (NPOS, NPOS), 1)
    pp = lax.broadcasted_iota(jnp.int32, (NPOS, NPOS), 0)
    shift = jnp.concatenate(
        [jnp.zeros((NPOS, 1), jnp.int32), rn64[:, 0:NPOS - 1]], axis=1)
    tslot = jnp.where(kk == pp, rp_col + 1,
                      N + 1 + jnp.where(kk < pp, rn64, shift))

    mtb = tlc == qlr
    mt_bf = jnp.where(mtb, 1.0, 0.0).astype(jnp.bfloat16)
    ioq = lax.broadcasted_iota(jnp.int32, (N, N), 0)
    ioq2 = lax.broadcasted_iota(jnp.int32, (N, N), 1)
    ltri_bf = jnp.where(ioq2 <= ioq, 1.0, 0.0).astype(jnp.bfloat16)
    ct = lax.dot_general(ltri_bf, mt_bf, (((1,), (0,)), ((), ())),
                         preferred_element_type=jnp.float32)
    cti = jnp.round(ct).astype(jnp.int32)

    qcol = lax.broadcasted_iota(jnp.int32, (N, NPOS), 0)
    slott = jnp.where(mtb, cti, N + qcol + 1 - cti)

    qi = lax.broadcasted_iota(jnp.int32, (1, N), 1)
    q2 = jnp.concatenate([(qi // 32).astype(jnp.bfloat16),
                          (qi % 32).astype(jnp.bfloat16)], axis=0)
    for p in range(0, NPOS, 2):
        sl2 = jnp.concatenate(
            [jnp.broadcast_to(slott[:, p:p + 1], (N, NPOS)),
             jnp.broadcast_to(slott[:, p + 1:p + 2], (N, NPOS))], axis=1)
        ts2 = jnp.concatenate(
            [tslot[p:p + 1, :], tslot[p + 1:p + 2, :]], axis=1)
        m2 = jnp.where(sl2 == ts2, 1.0, 0.0).astype(jnp.bfloat16)
        r2 = lax.dot_general(q2, m2, (((1,), (0,)), ((), ())),
                             preferred_element_type=jnp.float32)
        qsel2 = jnp.round(32.0 * r2[0:1, :] + r2[1:2, :]).astype(jnp.int32)
        fidx_ref[0, p:p + 1, :] = qsel2[:, 0:NPOS] + (b * NPOS + p) * N
        fidx_ref[0, p + 1:p + 2, :] = (
            qsel2[:, NPOS:2 * NPOS] + (b * NPOS + p + 1) * N)


def _stage1(qf, tf, qlc, tl3, tlc, qlr, tab):
    return pl.pallas_call(
        _stage1_body,
        grid=(B,),
        in_specs=[
            pl.BlockSpec((1, NPOS, D), lambda b: (b, 0, 0)),
            pl.BlockSpec((1, N, D), lambda b: (b, 0, 0)),
            pl.BlockSpec((1, NPOS, 128), lambda b: (b, 0, 0)),
            pl.BlockSpec((1, 1, N), lambda b: (b, 0, 0)),
            pl.BlockSpec((1, N, 128), lambda b: (b, 0, 0)),
            pl.BlockSpec((1, 1, NPOS), lambda b: (b, 0, 0)),
            pl.BlockSpec((N, 128), lambda b: (0, 0)),
        ],
        out_specs=[
            pl.BlockSpec((1, NPOS, N), lambda b: (b, 0, 0)),
            pl.BlockSpec((1, NPOS, NPOS), lambda b: (b, 0, 0)),
            pl.BlockSpec((1, 1, 128), lambda b: (b, 0, 0)),
        ],
        out_shape=[
            jax.ShapeDtypeStruct((B, NPOS, N), jnp.float32),
            jax.ShapeDtypeStruct((B, NPOS, NPOS), jnp.int32),
            jax.ShapeDtypeStruct((B, 1, 128), jnp.float32),
        ],
    )(qf, tf, qlc, tl3, tlc, qlr, tab)


def _sc_gather_body(s_hbm, fidx_hbm, out_hbm, iv, ov, sem):
    wid = lax.axis_index("s") * 2 + lax.axis_index("c")
    pltpu.sync_copy(fidx_hbm.at[wid], iv)
    copies = [pltpu.async_copy(s_hbm.at[iv.at[j]], ov.at[j], sem)
              for j in range(4)]
    for c in copies:
        c.wait()
    pltpu.sync_copy(ov, out_hbm.at[wid])


@functools.lru_cache(maxsize=1)
def _sc_gather_kernel():
    mesh = plsc.VectorSubcoreMesh(core_axis_name="c", subcore_axis_name="s")
    return pl.kernel(
        _sc_gather_body,
        mesh=mesh,
        out_type=jax.ShapeDtypeStruct((32, 4, 128), jnp.float32),
        scratch_types=[
            pltpu.VMEM((4, 128), jnp.int32),
            pltpu.VMEM((4, 128), jnp.float32),
            pltpu.SemaphoreType.DMA,
        ],
    )


def _stage2_body(lr_ref, ok_ref, loss_ref, out_ref):
    kk = lax.broadcasted_iota(jnp.int32, (NPOS, NPOS), 1)
    pp = lax.broadcasted_iota(jnp.int32, (NPOS, NPOS), 0)
    total = jnp.float32(0.0)
    for b in range(B):
        lr = lr_ref[b]
        okv = ok_ref[b]
        oks = jnp.max(okv)
        pred = okv[:, 0:NPOS] > 0.5
        gated = jnp.where(pred, lr * jnp.float32(1.0 / TEMP), jnp.float32(0.0))
        out_ref[b] = gated
        rmax = jnp.max(gated, axis=1, keepdims=True)
        e = jnp.exp(gated - rmax)
        lse = jnp.log(jnp.sum(e, axis=1, keepdims=True)) + rmax
        picked = jnp.sum(jnp.where(kk == pp, gated, 0.0), axis=1, keepdims=True)
        ce = jnp.sum(lse - picked) * jnp.float32(1.0 / NPOS)
        total = total + jnp.where(oks > 0.5, ce, jnp.float32(0.0))
    loss_ref[...] = jnp.full((1, 128), total, jnp.float32)


def _stage2(lraw, okf):
    return pl.pallas_call(
        _stage2_body,
        out_shape=[
            jax.ShapeDtypeStruct((1, 128), jnp.float32),
            jax.ShapeDtypeStruct((B, NPOS, NPOS), jnp.float32),
        ],
    )(lraw, okf)


def kernel(q_seed_features, q_seed_labels, t_seed_features, t_seed_labels,
           enc_inds_q, enc_inds_t, cl_loss_label, pc_q, pc_t):
    ql64 = q_seed_labels[:, :NPOS].astype(jnp.int32)
    tl = t_seed_labels.astype(jnp.int32)
    qlc = jnp.broadcast_to(ql64[:, :, None], (B, NPOS, 128))
    tlc = jnp.broadcast_to(tl[:, :, None], (B, N, 128))
    tl3 = tl.reshape(B, 1, N)
    qlr = ql64.reshape(B, 1, NPOS)
    tab = jnp.asarray(_COMB)

    s, fidx, okf = _stage1(q_seed_features, t_seed_features, qlc, tl3, tlc,
                           qlr, tab)

    lraw = _sc_gather_kernel()(s.reshape(B * NPOS * N), fidx.reshape(32, 4, 128))

    loss_v, out = _stage2(lraw.reshape(B, NPOS, NPOS), okf)
    return loss_v[0, 0], out


# speedup vs baseline: 76.8514x; 1.8205x over previous
import functools

import numpy as np
import jax
import jax.numpy as jnp
from jax import lax
from jax.experimental import pallas as pl
from jax.experimental.pallas import tpu as pltpu
from jax.experimental.pallas import tpu_sc as plsc

B, N, D = 4, 1024, 128
NPOS = 64
TEMP = 0.07

_rp = np.zeros(N, dtype=np.int64)
_rn = np.zeros((N, NPOS - 1), dtype=np.int64)
for _m in range(1, N):
    np.random.seed(0)
    _rp[_m] = np.random.choice(_m, 1)[0]
    _k = N - _m
    _rn[_m] = np.random.choice(_k, NPOS - 1, replace=_k < NPOS - 1)
_COMB = np.zeros((N, 128), dtype=np.float32)
_COMB[:, : NPOS - 1] = _rn
_COMB[:, NPOS - 1] = _rp
_COMBT_HI = np.ascontiguousarray((_COMB.T // 32).astype(np.float32))
_COMBT_LO = np.ascontiguousarray((_COMB.T % 32).astype(np.float32))


def _stage1_body(qf_ref, tf_ref, qlc_ref, tl3_ref, tlc_ref, qlr_ref,
                 tabt_hi_ref, tabt_lo_ref, s_ref, fidxt_ref, ok_ref):
    b = pl.program_id(0)
    qf = qf_ref[0]
    tf = tf_ref[0]
    qlc = qlc_ref[0][:, 0:1]
    tl = tl3_ref[0]
    tlc = tlc_ref[0][:, 0:1]
    qlr = qlr_ref[0]

    qn = qf / jnp.maximum(jnp.sqrt(jnp.sum(qf * qf, axis=1, keepdims=True)), 1e-12)
    tn = tf / jnp.maximum(jnp.sqrt(jnp.sum(tf * tf, axis=1, keepdims=True)), 1e-12)
    s_ref[0] = lax.dot_general(qn, tn, (((1,), (1,)), ((), ())),
                               preferred_element_type=jnp.float32,
                               precision=lax.Precision.HIGHEST)

    mask = qlc == tl
    mf = jnp.sum(mask.astype(jnp.float32), axis=1, keepdims=True)
    okv = jnp.where((mf >= 1.0) & (mf <= float(N - 1)), 1.0, 0.0)
    ok_ref[0] = jnp.full((1, 128), jnp.min(okv), jnp.float32)

    mask_bf = jnp.where(mask, 1.0, 0.0).astype(jnp.bfloat16)
    ioq = lax.broadcasted_iota(jnp.int32, (N, N), 0)
    ioq2 = lax.broadcasted_iota(jnp.int32, (N, N), 1)
    utri_bf = jnp.where(ioq <= ioq2, 1.0, 0.0).astype(jnp.bfloat16)
    c = lax.dot_general(mask_bf, utri_bf, (((1,), (0,)), ((), ())),
                        preferred_element_type=jnp.float32)
    ci = jnp.round(c).astype(jnp.int32)

    qrow = lax.broadcasted_iota(jnp.int32, (NPOS, N), 1)
    slot = jnp.where(mask, ci, N + qrow + 1 - ci)

    mtb = tlc == qlr
    mt_bf = jnp.where(mtb, 1.0, 0.0).astype(jnp.bfloat16)
    ones_bf = jnp.full((1, N), 1.0, jnp.bfloat16)
    m_row = lax.dot_general(ones_bf, mt_bf, (((1,), (0,)), ((), ())),
                            preferred_element_type=jnp.float32)
    mc_row = jnp.clip(jnp.round(m_row), 1.0, float(N - 1)).astype(jnp.int32)

    vio = lax.broadcasted_iota(jnp.int32, (N, NPOS), 0)
    oht_bf = jnp.where(vio == mc_row, 1.0, 0.0).astype(jnp.bfloat16)
    ch = lax.dot_general(tabt_hi_ref[...].astype(jnp.bfloat16), oht_bf,
                         (((1,), (0,)), ((), ())),
                         preferred_element_type=jnp.float32)
    cl = lax.dot_general(tabt_lo_ref[...].astype(jnp.bfloat16), oht_bf,
                         (((1,), (0,)), ((), ())),
                         preferred_element_type=jnp.float32)
    combt = jnp.round(32.0 * ch + cl).astype(jnp.int32)

    combt64 = combt[0:NPOS, :]
    rp_row = combt[NPOS - 1:NPOS, :]
    kk = lax.broadcasted_iota(jnp.int32, (NPOS, NPOS), 0)
    pp = lax.broadcasted_iota(jnp.int32, (NPOS, NPOS), 1)
    shift = jnp.concatenate(
        [jnp.zeros((1, NPOS), jnp.int32), combt64[0:NPOS - 1, :]], axis=0)
    tslott = jnp.where(kk == pp, rp_row + 1,
                       N + 1 + jnp.where(kk < pp, combt64, shift))

    qic = lax.broadcasted_iota(jnp.int32, (N, 1), 0)
    q2t = jnp.concatenate([(qic // 32).astype(jnp.bfloat16),
                           (qic % 32).astype(jnp.bfloat16)], axis=1)
    for p in range(NPOS):
        srow = slot[p:p + 1, :]
        tcol = tslott[:, p:p + 1]
        mkq = jnp.where(tcol == srow, 1.0, 0.0).astype(jnp.bfloat16)
        r = lax.dot_general(mkq, q2t, (((1,), (0,)), ((), ())),
                            preferred_element_type=jnp.float32)
        qsel = jnp.round(32.0 * r[:, 0:1] + r[:, 1:2]).astype(jnp.int32)
        fidxt_ref[0, :, p:p + 1] = qsel + (b * NPOS + p) * N


def _stage1(qf, tf, qlc, tl3, tlc, qlr, tabt_hi, tabt_lo):
    return pl.pallas_call(
        _stage1_body,
        grid=(B,),
        in_specs=[
            pl.BlockSpec((1, NPOS, D), lambda b: (b, 0, 0)),
            pl.BlockSpec((1, N, D), lambda b: (b, 0, 0)),
            pl.BlockSpec((1, NPOS, 128), lambda b: (b, 0, 0)),
            pl.BlockSpec((1, 1, N), lambda b: (b, 0, 0)),
            pl.BlockSpec((1, N, 128), lambda b: (b, 0, 0)),
            pl.BlockSpec((1, 1, NPOS), lambda b: (b, 0, 0)),
            pl.BlockSpec((128, N), lambda b: (0, 0)),
            pl.BlockSpec((128, N), lambda b: (0, 0)),
        ],
        out_specs=[
            pl.BlockSpec((1, NPOS, N), lambda b: (b, 0, 0)),
            pl.BlockSpec((1, NPOS, NPOS), lambda b: (b, 0, 0)),
            pl.BlockSpec((1, 1, 128), lambda b: (b, 0, 0)),
        ],
        out_shape=[
            jax.ShapeDtypeStruct((B, NPOS, N), jnp.float32),
            jax.ShapeDtypeStruct((B, NPOS, NPOS), jnp.int32),
            jax.ShapeDtypeStruct((B, 1, 128), jnp.float32),
        ],
    )(qf, tf, qlc, tl3, tlc, qlr, tabt_hi, tabt_lo)


def _sc_gather_body(s_hbm, fidxt_hbm, out_hbm, pv, iv, ov, sem):
    wid = lax.axis_index("s") * 2 + lax.axis_index("c")
    b = wid // 8
    lane = lax.iota(jnp.int32, 16) * NPOS
    for j in range(4):
        for cch in range(8):
            t = 2 * j + (cch // 4)
            k0 = (cch % 4) * 16
            p = (wid % 8) * 8 + t
            pv[j, pl.ds(cch * 16, 16)] = lane + (b * 4096 + k0 * NPOS + p)
    l1 = [pltpu.async_copy(fidxt_hbm.at[pv.at[j]], iv.at[j], sem)
          for j in range(4)]
    for cp in l1:
        cp.wait()
    l2 = [pltpu.async_copy(s_hbm.at[iv.at[j]], ov.at[j], sem)
          for j in range(4)]
    for cp in l2:
        cp.wait()
    pltpu.sync_copy(ov, out_hbm.at[wid])


@functools.lru_cache(maxsize=1)
def _sc_gather_kernel():
    mesh = plsc.VectorSubcoreMesh(core_axis_name="c", subcore_axis_name="s")
    return pl.kernel(
        _sc_gather_body,
        mesh=mesh,
        out_type=jax.ShapeDtypeStruct((32, 4, 128), jnp.float32),
        scratch_types=[
            pltpu.VMEM((4, 128), jnp.int32),
            pltpu.VMEM((4, 128), jnp.int32),
            pltpu.VMEM((4, 128), jnp.float32),
            pltpu.SemaphoreType.DMA,
        ],
    )


def _stage2_body(lr_ref, ok_ref, loss_ref, out_ref):
    kk = lax.broadcasted_iota(jnp.int32, (NPOS, NPOS), 1)
    pp = lax.broadcasted_iota(jnp.int32, (NPOS, NPOS), 0)
    total = jnp.float32(0.0)
    for b in range(B):
        lr = lr_ref[b]
        okv = ok_ref[b]
        oks = jnp.max(okv)
        pred = okv[:, 0:NPOS] > 0.5
        gated = jnp.where(pred, lr * jnp.float32(1.0 / TEMP), jnp.float32(0.0))
        out_ref[b] = gated
        rmax = jnp.max(gated, axis=1, keepdims=True)
        e = jnp.exp(gated - rmax)
        lse = jnp.log(jnp.sum(e, axis=1, keepdims=True)) + rmax
        picked = jnp.sum(jnp.where(kk == pp, gated, 0.0), axis=1, keepdims=True)
        ce = jnp.sum(lse - picked) * jnp.float32(1.0 / NPOS)
        total = total + jnp.where(oks > 0.5, ce, jnp.float32(0.0))
    loss_ref[...] = jnp.full((1, 128), total, jnp.float32)


def _stage2(lraw, okf):
    return pl.pallas_call(
        _stage2_body,
        out_shape=[
            jax.ShapeDtypeStruct((1, 128), jnp.float32),
            jax.ShapeDtypeStruct((B, NPOS, NPOS), jnp.float32),
        ],
    )(lraw, okf)


def kernel(q_seed_features, q_seed_labels, t_seed_features, t_seed_labels,
           enc_inds_q, enc_inds_t, cl_loss_label, pc_q, pc_t):
    ql64 = q_seed_labels[:, :NPOS].astype(jnp.int32)
    tl = t_seed_labels.astype(jnp.int32)
    qlc = jnp.broadcast_to(ql64[:, :, None], (B, NPOS, 128))
    tlc = jnp.broadcast_to(tl[:, :, None], (B, N, 128))
    tl3 = tl.reshape(B, 1, N)
    qlr = ql64.reshape(B, 1, NPOS)
    tabt_hi = jnp.asarray(_COMBT_HI)
    tabt_lo = jnp.asarray(_COMBT_LO)

    s, fidxt, okf = _stage1(q_seed_features, t_seed_features, qlc, tl3, tlc,
                            qlr, tabt_hi, tabt_lo)

    lraw = _sc_gather_kernel()(s.reshape(B * NPOS * N),
                               fidxt.reshape(B * NPOS * NPOS))

    loss_v, out = _stage2(lraw.reshape(B, NPOS, NPOS), okf)
    return loss_v[0, 0], out
